# baseline (speedup 1.0000x reference)
import functools

import jax
import jax.numpy as jnp
import numpy as np
from jax import lax
from jax.experimental import pallas as pl
from jax.experimental.pallas import tpu as pltpu
from jax.experimental.pallas import tpu_sc as plsc

NUM_CPNTS = 16
IN_CHANNELS = 384
BATCH = 64
ROW = 1024
DELTA_T = 2.0 / (NUM_CPNTS - 1)
CP = np.linspace(-1.0, 1.0, NUM_CPNTS).astype(np.float32)
INV_D = np.float32(7.5)
NTAB = NUM_CPNTS + 1

NC, NS, L = 2, 16, 16
NW = NC * NS
N_ROWS = BATCH * IN_CHANNELS
ROWS_PER_W = N_ROWS // NW
CHUNK_ROWS = 16
CHUNK_EL = CHUNK_ROWS * ROW
N_CHUNKS = ROWS_PER_W // CHUNK_ROWS
NBUF = 3
TAB = IN_CHANNELS * NTAB


def _body(x2, st, bs, out, ptab, qtab, stab, btab, inb, outb,
          sin0, sin1, sin2, sout0, sout1, sout2):
    cid = lax.axis_index("c")
    sid = lax.axis_index("s")
    wid = sid * NC + cid
    row0 = wid * ROWS_PER_W

    def _in_copy0(t, p, buf, sem):
        return pltpu.make_async_copy(
            x2.at[pl.ds(row0 + t * CHUNK_ROWS, CHUNK_ROWS), :], buf, sem)

    _in_copy0(0, 0, inb.at[0], sin0).start()
    _in_copy0(1, 1, inb.at[1], sin1).start()

    pltpu.sync_copy(st, stab)
    pltpu.sync_copy(bs, btab)

    lane17 = lax.iota(jnp.int32, L) * NTAB

    def build(g, carry):
        cb = g * L
        bias_v = btab[pl.ds(cb, L)]
        sidx = lane17 + cb * NTAB
        acc = bias_v
        s0 = stab[pl.ds(cb, L)]
        plsc.store_scatter(ptab, [sidx], s0)
        plsc.store_scatter(qtab, [sidx], acc - CP[0] * s0)
        for k in range(1, NUM_CPNTS + 1):
            sk = stab[pl.ds(k * IN_CHANNELS + cb, L)]
            plsc.store_scatter(ptab, [sidx + k], sk)
            plsc.store_scatter(qtab, [sidx + k], acc - CP[k - 1] * sk)
            if k <= NUM_CPNTS - 1:
                acc = acc + np.float32(DELTA_T) * sk
        return carry

    lax.fori_loop(0, IN_CHANNELS // L, build, 0)

    sins = [sin0, sin1, sin2]
    souts = [sout0, sout1, sout2]

    def in_copy(t, p):
        return pltpu.make_async_copy(
            x2.at[pl.ds(row0 + t * CHUNK_ROWS, CHUNK_ROWS), :],
            inb.at[p], sins[p])

    def out_copy(t, p):
        return pltpu.make_async_copy(
            outb.at[p],
            out.at[pl.ds(row0 + t * CHUNK_ROWS, CHUNK_ROWS), :], souts[p])

    def process(t, p):
        @pl.when(t + 2 < N_CHUNKS)
        def _():
            in_copy(t + 2, (p + 2) % NBUF).start()

        in_copy(t, p).wait()

        @pl.when(t >= NBUF)
        def _():
            out_copy(t, p).wait()

        cbase = lax.rem(t * CHUNK_ROWS, IN_CHANNELS) * NTAB

        @plsc.parallel_loop(0, CHUNK_EL // L, 1, unroll=4)
        def _vloop(k):
            r = lax.shift_right_logical(k, 6)
            col = lax.shift_left(k & (ROW // L - 1), 4)
            coff = cbase + r * NTAB
            v = inb[p, r, pl.ds(col, L)]
            u = v * INV_D + np.float32(8.5)
            u = jnp.minimum(jnp.maximum(u, np.float32(0.0)),
                            np.float32(16.5))
            idx = u.astype(jnp.int32) + coff
            pv = plsc.load_gather(ptab, [idx])
            qv = plsc.load_gather(qtab, [idx])
            outb[p, r, pl.ds(col, L)] = v * pv + qv

        out_copy(t, p).start()

    def chunk_trip(i, carry):
        for p in range(NBUF):
            process(i * NBUF + p, p)
        return carry

    lax.fori_loop(0, N_CHUNKS // NBUF, chunk_trip, 0)

    out_copy(N_CHUNKS - 3, 0).wait()
    out_copy(N_CHUNKS - 2, 1).wait()
    out_copy(N_CHUNKS - 1, 2).wait()


_pwl = functools.partial(
    pl.kernel,
    out_type=jax.ShapeDtypeStruct((N_ROWS, ROW), jnp.float32),
    mesh=plsc.VectorSubcoreMesh(core_axis_name="c", subcore_axis_name="s"),
    compiler_params=pltpu.CompilerParams(needs_layout_passes=False,
                                         use_tc_tiling_on_sc=True),
    scratch_types=[
        pltpu.VMEM((TAB,), jnp.float32),
        pltpu.VMEM((TAB,), jnp.float32),
        pltpu.VMEM((NTAB * IN_CHANNELS,), jnp.float32),
        pltpu.VMEM((IN_CHANNELS,), jnp.float32),
        pltpu.VMEM((NBUF, CHUNK_ROWS, ROW), jnp.float32),
        pltpu.VMEM((NBUF, CHUNK_ROWS, ROW), jnp.float32),
        pltpu.SemaphoreType.DMA,
        pltpu.SemaphoreType.DMA,
        pltpu.SemaphoreType.DMA,
        pltpu.SemaphoreType.DMA,
        pltpu.SemaphoreType.DMA,
        pltpu.SemaphoreType.DMA,
    ],
)(_body)


def kernel(x, slopes, biases):
    x2 = x.reshape(N_ROWS, ROW)
    st = slopes.T.reshape(-1)
    out = _pwl(x2, st, biases)
    return out.reshape(x.shape)


# speedup vs baseline: 1.0432x; 1.0432x over previous
import functools

import jax
import jax.numpy as jnp
import numpy as np
from jax import lax
from jax.experimental import pallas as pl
from jax.experimental.pallas import tpu as pltpu
from jax.experimental.pallas import tpu_sc as plsc

NUM_CPNTS = 16
IN_CHANNELS = 384
BATCH = 64
ROW = 1024
DELTA_T = 2.0 / (NUM_CPNTS - 1)
CP = np.linspace(-1.0, 1.0, NUM_CPNTS).astype(np.float32)
INV_D = np.float32(7.5)
NTAB = NUM_CPNTS + 1

NC, NS, L = 2, 16, 16
NW = NC * NS
N_ROWS = BATCH * IN_CHANNELS
ROWS_PER_W = N_ROWS // NW
CHUNK_ROWS = 16
CHUNK_EL = CHUNK_ROWS * ROW
N_CHUNKS = ROWS_PER_W // CHUNK_ROWS
NBUF = 3
TAB = IN_CHANNELS * NTAB


def _body(x2, st, bs, out, ptab, qtab, stab, btab, inb, outb,
          sin0, sin1, sin2, sout0, sout1, sout2):
    cid = lax.axis_index("c")
    sid = lax.axis_index("s")
    wid = sid * NC + cid
    row0 = wid * ROWS_PER_W

    def _in_copy0(t, p, buf, sem):
        return pltpu.make_async_copy(
            x2.at[pl.ds(row0 + t * CHUNK_ROWS, CHUNK_ROWS), :], buf, sem)

    _in_copy0(0, 0, inb.at[0], sin0).start()
    _in_copy0(1, 1, inb.at[1], sin1).start()

    pltpu.sync_copy(st, stab)
    pltpu.sync_copy(bs, btab)

    lane17 = lax.iota(jnp.int32, L) * NTAB

    def build(g, carry):
        cb = g * L
        bias_v = btab[pl.ds(cb, L)]
        sidx = lane17 + cb * NTAB
        acc = bias_v
        s0 = stab[pl.ds(cb, L)]
        plsc.store_scatter(ptab, [sidx], s0)
        plsc.store_scatter(qtab, [sidx], acc - CP[0] * s0)
        for k in range(1, NUM_CPNTS + 1):
            sk = stab[pl.ds(k * IN_CHANNELS + cb, L)]
            plsc.store_scatter(ptab, [sidx + k], sk)
            plsc.store_scatter(qtab, [sidx + k], acc - CP[k - 1] * sk)
            if k <= NUM_CPNTS - 1:
                acc = acc + np.float32(DELTA_T) * sk
        return carry

    lax.fori_loop(0, IN_CHANNELS // L, build, 0)

    sins = [sin0, sin1, sin2]
    souts = [sout0, sout1, sout2]

    def in_copy(t, p):
        return pltpu.make_async_copy(
            x2.at[pl.ds(row0 + t * CHUNK_ROWS, CHUNK_ROWS), :],
            inb.at[p], sins[p])

    def out_copy(t, p):
        return pltpu.make_async_copy(
            outb.at[p],
            out.at[pl.ds(row0 + t * CHUNK_ROWS, CHUNK_ROWS), :], souts[p])

    def process(t, p):
        @pl.when(t + 2 < N_CHUNKS)
        def _():
            in_copy(t + 2, (p + 2) % NBUF).start()

        in_copy(t, p).wait()

        @pl.when(t >= NBUF)
        def _():
            out_copy(t, p).wait()

        cbase = lax.rem(t * CHUNK_ROWS, IN_CHANNELS) * NTAB

        @plsc.parallel_loop(0, CHUNK_EL // L, 1, unroll=8)
        def _vloop(k):
            r = lax.shift_right_logical(k, 6)
            col = lax.shift_left(k & (ROW // L - 1), 4)
            coff = cbase + r * NTAB
            v = inb[p, r, pl.ds(col, L)]
            u = v * INV_D + np.float32(8.5)
            u = jnp.minimum(jnp.maximum(u, np.float32(0.0)),
                            np.float32(16.5))
            idx = u.astype(jnp.int32) + coff
            pv = plsc.load_gather(ptab, [idx])
            qv = plsc.load_gather(qtab, [idx])
            outb[p, r, pl.ds(col, L)] = v * pv + qv

        out_copy(t, p).start()

    def chunk_trip(i, carry):
        for p in range(NBUF):
            process(i * NBUF + p, p)
        return carry

    lax.fori_loop(0, N_CHUNKS // NBUF, chunk_trip, 0)

    out_copy(N_CHUNKS - 3, 0).wait()
    out_copy(N_CHUNKS - 2, 1).wait()
    out_copy(N_CHUNKS - 1, 2).wait()


_pwl = functools.partial(
    pl.kernel,
    out_type=jax.ShapeDtypeStruct((N_ROWS, ROW), jnp.float32),
    mesh=plsc.VectorSubcoreMesh(core_axis_name="c", subcore_axis_name="s"),
    compiler_params=pltpu.CompilerParams(needs_layout_passes=False,
                                         use_tc_tiling_on_sc=True),
    scratch_types=[
        pltpu.VMEM((TAB,), jnp.float32),
        pltpu.VMEM((TAB,), jnp.float32),
        pltpu.VMEM((NTAB * IN_CHANNELS,), jnp.float32),
        pltpu.VMEM((IN_CHANNELS,), jnp.float32),
        pltpu.VMEM((NBUF, CHUNK_ROWS, ROW), jnp.float32),
        pltpu.VMEM((NBUF, CHUNK_ROWS, ROW), jnp.float32),
        pltpu.SemaphoreType.DMA,
        pltpu.SemaphoreType.DMA,
        pltpu.SemaphoreType.DMA,
        pltpu.SemaphoreType.DMA,
        pltpu.SemaphoreType.DMA,
        pltpu.SemaphoreType.DMA,
    ],
)(_body)


def kernel(x, slopes, biases):
    x2 = x.reshape(N_ROWS, ROW)
    st = slopes.T.reshape(-1)
    out = _pwl(x2, st, biases)
    return out.reshape(x.shape)
